# Initial kernel scaffold; baseline (speedup 1.0000x reference)
import jax
import jax.numpy as jnp
from jax.experimental import pallas as pl


def kernel(x, edge_index, edge_attr, u, edge_batch, node_batch, num_nodes, num_edges, We, be, Wn, bn, Wg, bg, na_W1, na_b1, na_w2, na_W3, na_b3, gn_W1, gn_b1, gn_w2, ge_W1, ge_b1, ge_w2):
    raise NotImplementedError("write your pallas kernel here")


# speedup vs baseline: 5.2614x; 5.2614x over previous
import functools

import jax
import jax.numpy as jnp
import numpy as np
from jax import lax
from jax.experimental import pallas as pl
from jax.experimental.pallas import tpu as pltpu
from jax.experimental.pallas import tpu_sc as plsc

F32 = jnp.float32

_NC = 2
_NS = 16
_NW = _NC * _NS
_CK = 128
_SK = 80


def _leaky(v):
    return jnp.where(v >= 0, v, 0.01 * v)


def _dot(a, b):
    return jax.lax.dot_general(a, b, (((1,), (0,)), ((), ())),
                               preferred_element_type=F32)


def _dot_t(a, b):
    return jax.lax.dot_general(a, b, (((0,), (0,)), ((), ())),
                               preferred_element_type=F32)


def _gather_call(N, E, D):
    mesh = plsc.VectorSubcoreMesh(core_axis_name="c", subcore_axis_name="s",
                                  num_cores=_NC, num_subcores=_NS)
    nch = E // _CK
    iters = (nch + _NW - 1) // _NW

    @functools.partial(
        pl.kernel,
        out_type=(jax.ShapeDtypeStruct((E, D), F32),
                  jax.ShapeDtypeStruct((E, D), F32)),
        mesh=mesh,
        scratch_types=[pltpu.VMEM((_CK,), jnp.int32),
                       pltpu.VMEM((_CK, D), F32),
                       pltpu.SemaphoreType.DMA],
    )
    def gk(x_hbm, row_hbm, col_hbm, xr_out, xc_out, idx_v, rows_v, sem):
        c = lax.axis_index("c")
        s = lax.axis_index("s")
        wid = s * _NC + c

        @pl.loop(0, iters)
        def _(t):
            cid = wid + t * _NW

            @pl.when(cid < nch)
            def _():
                base = cid * _CK
                pltpu.sync_copy(row_hbm.at[pl.ds(base, _CK)], idx_v)
                pltpu.async_copy(x_hbm.at[idx_v], rows_v, sem).wait()
                pltpu.sync_copy(rows_v, xr_out.at[pl.ds(base, _CK)])
                pltpu.sync_copy(col_hbm.at[pl.ds(base, _CK)], idx_v)
                pltpu.async_copy(x_hbm.at[idx_v], rows_v, sem).wait()
                pltpu.sync_copy(rows_v, xc_out.at[pl.ds(base, _CK)])

    return gk


def _scatter_call(N, E, D):
    mesh = plsc.VectorSubcoreMesh(core_axis_name="c", subcore_axis_name="s",
                                  num_cores=_NC, num_subcores=_NS)
    nch = E // _SK
    iters = nch // _NS
    nzch = N // _SK
    ziters = (nzch + _NS - 1) // _NS

    @functools.partial(
        pl.kernel,
        out_type=(jax.ShapeDtypeStruct((2, N, D), F32),
                  jax.ShapeDtypeStruct((2, N, D), F32)),
        mesh=mesh,
        scratch_types=[pltpu.VMEM((_SK,), jnp.int32),
                       pltpu.VMEM((_SK, D), F32),
                       pltpu.VMEM_SHARED((N, D), F32)],
    )
    def sk(zsv, zse, zrv, zre, row_hbm, col_hbm, zv_hbm,
           outv, oute, idx_v, buf, acc):
        c = lax.axis_index("c")
        s = lax.axis_index("s")

        def zero_acc():
            pltpu.sync_copy(zv_hbm, buf)

            @pl.loop(0, ziters)
            def _(t):
                cid = s + t * _NS

                @pl.when(cid < nzch)
                def _():
                    pltpu.sync_copy(buf, acc.at[pl.ds(cid * _SK, _SK)])

            plsc.subcore_barrier()

        def scatter(key_hbm, dat_hbm):
            @pl.loop(0, iters)
            def _(t):
                base = (s + t * _NS) * _SK
                pltpu.sync_copy(key_hbm.at[pl.ds(base, _SK)], idx_v)
                pltpu.sync_copy(dat_hbm.at[pl.ds(base, _SK)], buf)
                pltpu.sync_copy(buf, acc.at[idx_v], add=True)

            plsc.subcore_barrier()

        def drain(out_hbm):
            @pl.loop(0, ziters)
            def _(t):
                cid = s + t * _NS

                @pl.when(cid < nzch)
                def _():
                    r0 = cid * _SK
                    pltpu.sync_copy(acc.at[pl.ds(r0, _SK)], buf)
                    pltpu.sync_copy(buf, out_hbm.at[c, pl.ds(r0, _SK)])

            plsc.subcore_barrier()

        def both_phases(key_hbm, v_hbm, e_hbm):
            zero_acc()
            scatter(key_hbm, v_hbm)
            drain(outv)
            zero_acc()
            scatter(key_hbm, e_hbm)
            drain(oute)

        @pl.when(c == 0)
        def _():
            both_phases(row_hbm, zsv, zse)

        @pl.when(c == 1)
        def _():
            both_phases(col_hbm, zrv, zre)

    return sk


def _edge_body(ng, ea_ref, xr_ref, xc_ref, we0, we1, we2, cwe,
               a1, a2, a3, b1, c1, c2, b3, w2m, smat, mask16, p16, g2t, cg,
               w2g, ea2_ref, zsv_ref, zse_ref, zrv_ref, zre_ref,
               sze_ref, sse_ref, accz, accs):
    i = pl.program_id(0)

    @pl.when(i == 0)
    def _():
        accz[...] = jnp.zeros_like(accz)
        accs[...] = jnp.zeros_like(accs)

    ea = ea_ref[...]
    xr = xr_ref[...]
    xc = xc_ref[...]
    ea2 = _dot(ea, we0[...]) + _dot(xr, we1[...]) + _dot(xc, we2[...]) + cwe[...]
    ea2_ref[...] = ea2

    ea2a3 = _dot(ea2, a3[...]) + b1[...]
    ea2c2 = _dot(ea2, c2[...]) + b3[...]

    def side(q, kv, v_ref, e_ref):
        h = _leaky(_dot(q, a1[...]) + _dot(kv, a2[...]) + ea2a3)
        e = jnp.exp(_dot(h, w2m[...])) * mask16[...]
        v = _dot(kv, c1[...]) + ea2c2
        v_ref[...] = _dot(e, smat[...]) * v
        e_ref[...] = _dot(e, p16[...])

    side(xr, xc, zsv_ref, zse_ref)
    side(xc, xr, zrv_ref, zre_ref)

    hg = _leaky(_dot(ea2, g2t[...]) + cg[...])
    eg = jnp.exp(_dot(hg, w2g[...])) * mask16[...]
    accz[...] += _dot_t(eg, ea2)
    accs[...] += jnp.sum(eg, axis=0, keepdims=True)

    @pl.when(i == ng - 1)
    def _():
        sze_ref[...] = accz[...]
        sse_ref[...] = accs[...]


def _edge_call(E, D, EB):
    ng = E // EB
    blk = lambda r, c: pl.BlockSpec((r, c), lambda i: (i, 0))
    full = lambda r, c: pl.BlockSpec((r, c), lambda i: (0, 0))
    return pl.pallas_call(
        functools.partial(_edge_body, ng),
        grid=(ng,),
        in_specs=[blk(EB, D), blk(EB, D), blk(EB, D),
                  full(D, D), full(D, D), full(D, D), full(1, D),
                  full(D, D), full(D, D), full(D, D), full(1, D),
                  full(D, D), full(D, D), full(1, D),
                  full(D, 16), full(16, D), full(1, 16), full(16, D),
                  full(D, D), full(1, D), full(D, 16)],
        out_specs=[blk(EB, D), blk(EB, D), blk(EB, D), blk(EB, D),
                   blk(EB, D), full(16, D), full(1, 16)],
        out_shape=[jax.ShapeDtypeStruct((E, D), F32),
                   jax.ShapeDtypeStruct((E, D), F32),
                   jax.ShapeDtypeStruct((E, D), F32),
                   jax.ShapeDtypeStruct((E, D), F32),
                   jax.ShapeDtypeStruct((E, D), F32),
                   jax.ShapeDtypeStruct((16, D), F32),
                   jax.ShapeDtypeStruct((1, 16), F32)],
        scratch_shapes=[pltpu.VMEM((16, D), F32), pltpu.VMEM((1, 16), F32)],
        compiler_params=pltpu.CompilerParams(
            dimension_semantics=("arbitrary",)),
    )


def _node_body(ng, x_ref, sv_ref, se_ref, rv_ref, re_ref,
               wn1, wn2, wn3, cwn, gn2, cgn, w2n, smat, smat128, mask16,
               sze_ref, sse_ref, u_ref, wg1, wg2, wg3, bg,
               x2_ref, u2_ref, accz, accs):
    i = pl.program_id(0)

    @pl.when(i == 0)
    def _():
        accz[...] = jnp.zeros_like(accz)
        accs[...] = jnp.zeros_like(accs)

    sm = smat[...]
    sm128 = smat128[...]
    sent = sv_ref[...] / (_dot(se_ref[...], sm128) + 1e-16)
    recv = rv_ref[...] / (_dot(re_ref[...], sm128) + 1e-16)
    x2 = (_dot(x_ref[...], wn1[...]) + _dot(sent, wn2[...])
          + _dot(recv, wn3[...]) + cwn[...])
    x2_ref[...] = x2

    hn = _leaky(_dot(x2, gn2[...]) + cgn[...])
    en = jnp.exp(_dot(hn, w2n[...])) * mask16[...]
    accz[...] += _dot_t(en, x2)
    accs[...] += jnp.sum(en, axis=0, keepdims=True)

    @pl.when(i == ng - 1)
    def _():
        naz = jnp.sum(accz[...] * sm, axis=0, keepdims=True)
        nad = _dot(accs[...], sm) + 1e-16
        na = naz / nad
        eaz = jnp.sum(sze_ref[...] * sm, axis=0, keepdims=True)
        ead = _dot(sse_ref[...], sm) + 1e-16
        eav = eaz / ead
        u2_ref[...] = (_dot(u_ref[...], wg1[...]) + _dot(na, wg2[...])
                       + _dot(eav, wg3[...]) + bg[...])


def _node_call(N, D, NB):
    ng = N // NB
    blk = lambda r, c: pl.BlockSpec((r, c), lambda i: (i, 0))
    full = lambda r, c: pl.BlockSpec((r, c), lambda i: (0, 0))
    return pl.pallas_call(
        functools.partial(_node_body, ng),
        grid=(ng,),
        in_specs=[blk(NB, D), blk(NB, D), blk(NB, D), blk(NB, D), blk(NB, D),
                  full(D, D), full(D, D), full(D, D), full(1, D),
                  full(D, D), full(1, D), full(D, 16), full(16, D),
                  full(D, D), full(1, 16), full(16, D), full(1, 16),
                  full(1, D), full(D, D), full(D, D), full(D, D),
                  full(1, D)],
        out_specs=[blk(NB, D), full(1, D)],
        out_shape=[jax.ShapeDtypeStruct((N, D), F32),
                   jax.ShapeDtypeStruct((1, D), F32)],
        scratch_shapes=[pltpu.VMEM((16, D), F32), pltpu.VMEM((1, 16), F32)],
        compiler_params=pltpu.CompilerParams(
            dimension_semantics=("arbitrary",)),
    )


def _headmats(D, H, w2):
    HD = D // H
    w2m = jnp.zeros((D, 16), F32)
    for h in range(H):
        w2m = w2m.at[h * HD:(h + 1) * HD, h].set(w2[h])
    return w2m


def kernel(x, edge_index, edge_attr, u, edge_batch, node_batch, num_nodes,
           num_edges, We, be, Wn, bn, Wg, bg, na_W1, na_b1, na_w2, na_W3,
           na_b3, gn_W1, gn_b1, gn_w2, ge_W1, ge_b1, ge_w2):
    N, D = x.shape
    E = edge_attr.shape[0]
    H = na_w2.shape[0]
    HD = D // H

    row = edge_index[0]
    col = edge_index[1]

    sp = np.zeros((16, D), np.float32)
    for h in range(H):
        sp[h, h * HD:(h + 1) * HD] = 1.0
    smat = jnp.asarray(sp)
    smat128 = jnp.concatenate([smat, jnp.zeros((D - 16, D), F32)], axis=0)
    p16 = jnp.concatenate([jnp.eye(16, dtype=F32),
                           jnp.zeros((16, D - 16), F32)], axis=1)
    mask16 = jnp.asarray((np.arange(16) < H).astype(np.float32))[None, :]

    def splitT(W, k):
        return [W[:, i * D:(i + 1) * D].T for i in range(k)]

    we = splitT(We, 4)
    cwe = (u @ we[3] + be).reshape(1, D)
    a1, a2, a3 = splitT(na_W1, 3)
    c1, c2 = splitT(na_W3, 2)
    w2m = _headmats(D, H, na_w2)
    g1, g2 = splitT(ge_W1, 2)
    cg = (u @ g1 + ge_b1).reshape(1, D)
    w2g = _headmats(D, H, ge_w2)

    xr, xc = _gather_call(N, E, D)(x, row, col)

    EB = 640
    ea2, zsv, zse, zrv, zre, sze, sse = _edge_call(E, D, EB)(
        edge_attr, xr, xc, we[0], we[1], we[2], cwe,
        a1, a2, a3, na_b1.reshape(1, D), c1, c2, na_b3.reshape(1, D),
        w2m, smat, mask16, p16, g2, cg, w2g)

    zv0 = jnp.zeros((_SK, D), F32)
    accv, acce = _scatter_call(N, E, D)(zsv, zse, zrv, zre, row, col, zv0)

    wn1, wn2, wn3, wn4 = splitT(Wn, 4)
    cwn = (u @ wn4 + bn).reshape(1, D)
    gn1, gn2 = splitT(gn_W1, 2)
    cgn = (u @ gn1 + gn_b1).reshape(1, D)
    w2n = _headmats(D, H, gn_w2)
    wg1, wg2, wg3 = splitT(Wg, 3)

    NB = 1000
    x2, u2 = _node_call(N, D, NB)(
        x, accv[0], acce[0], accv[1], acce[1],
        wn1, wn2, wn3, cwn, gn2, cgn, w2n, smat, smat128, mask16,
        sze, sse, u.reshape(1, D), wg1, wg2, wg3, bg.reshape(1, D))

    return (x2, ea2, u2)


# speedup vs baseline: 6.9645x; 1.3237x over previous
import functools

import jax
import jax.numpy as jnp
import numpy as np
from jax import lax
from jax.experimental import pallas as pl
from jax.experimental.pallas import tpu as pltpu
from jax.experimental.pallas import tpu_sc as plsc

F32 = jnp.float32

_NC = 2
_NS = 16
_CK = 128
_SK = 80


def _leaky(v):
    return jnp.where(v >= 0, v, 0.01 * v)


def _dot(a, b):
    return jax.lax.dot_general(a, b, (((1,), (0,)), ((), ())),
                               preferred_element_type=F32)


def _dot_t(a, b):
    return jax.lax.dot_general(a, b, (((0,), (0,)), ((), ())),
                               preferred_element_type=F32)


def _sc_mesh():
    return plsc.VectorSubcoreMesh(core_axis_name="c", subcore_axis_name="s",
                                  num_cores=_NC, num_subcores=_NS)


def _gather_call(N, E, D):
    nch = E // _CK
    iters = (nch + _NS - 1) // _NS

    @functools.partial(
        pl.kernel,
        out_type=(jax.ShapeDtypeStruct((E, D), F32),
                  jax.ShapeDtypeStruct((E, D), F32)),
        mesh=_sc_mesh(),
        scratch_types=[pltpu.VMEM((2, _CK), jnp.int32),
                       pltpu.VMEM((2, _CK, D), F32),
                       pltpu.SemaphoreType.DMA],
    )
    def gk(x_hbm, row_hbm, col_hbm, xr_out, xc_out, idx2, rows2, sem):
        c = lax.axis_index("c")
        s = lax.axis_index("s")

        def one_side(key_hbm, out_hbm):
            def start(t):
                cid = s + t * _NS

                @pl.when(cid < nch)
                def _():
                    slot = jnp.bitwise_and(t, 1)
                    base = cid * _CK
                    pltpu.sync_copy(key_hbm.at[pl.ds(base, _CK)],
                                    idx2.at[slot])
                    pltpu.async_copy(x_hbm.at[idx2.at[slot]],
                                     rows2.at[slot], sem)

            start(0)

            @pl.loop(0, iters)
            def _(t):
                start(t + 1)
                cid = s + t * _NS

                @pl.when(cid < nch)
                def _():
                    slot = jnp.bitwise_and(t, 1)
                    pltpu.make_async_copy(x_hbm.at[idx2.at[slot]],
                                          rows2.at[slot], sem).wait()
                    pltpu.sync_copy(rows2.at[slot],
                                    out_hbm.at[pl.ds(cid * _CK, _CK)])

        @pl.when(c == 0)
        def _():
            one_side(row_hbm, xr_out)

        @pl.when(c == 1)
        def _():
            one_side(col_hbm, xc_out)

    return gk


def _scatter_call(N, E, D):
    nch = E // _SK
    iters = nch // _NS
    nzch = N // _SK
    ziters = (nzch + _NS - 1) // _NS

    @functools.partial(
        pl.kernel,
        out_type=(jax.ShapeDtypeStruct((2, N, D), F32),
                  jax.ShapeDtypeStruct((2, N, D), F32)),
        mesh=_sc_mesh(),
        scratch_types=[pltpu.VMEM((2, _SK), jnp.int32),
                       pltpu.VMEM((2, _SK, D), F32),
                       pltpu.SemaphoreType.DMA,
                       pltpu.VMEM_SHARED((N, D), F32)],
    )
    def sk(zsv, zse, zrv, zre, row_hbm, col_hbm, zv_hbm,
           outv, oute, idx2, buf2, sem, acc):
        c = lax.axis_index("c")
        s = lax.axis_index("s")

        def zero_acc():
            pltpu.sync_copy(zv_hbm, buf2.at[0])

            @pl.loop(0, ziters)
            def _(t):
                cid = s + t * _NS

                @pl.when(cid < nzch)
                def _():
                    pltpu.sync_copy(buf2.at[0],
                                    acc.at[pl.ds(cid * _SK, _SK)])

            plsc.subcore_barrier()

        def scatter(key_hbm, dat_hbm):
            def start(t):
                @pl.when(t < iters)
                def _():
                    slot = jnp.bitwise_and(t, 1)
                    base = (s + t * _NS) * _SK
                    pltpu.sync_copy(key_hbm.at[pl.ds(base, _SK)],
                                    idx2.at[slot])
                    pltpu.async_copy(dat_hbm.at[pl.ds(base, _SK)],
                                     buf2.at[slot], sem)

            start(0)

            @pl.loop(0, iters)
            def _(t):
                start(t + 1)
                slot = jnp.bitwise_and(t, 1)
                base = (s + t * _NS) * _SK
                pltpu.make_async_copy(dat_hbm.at[pl.ds(base, _SK)],
                                      buf2.at[slot], sem).wait()
                pltpu.sync_copy(buf2.at[slot], acc.at[idx2.at[slot]],
                                add=True)

            plsc.subcore_barrier()

        def drain(out_hbm):
            @pl.loop(0, ziters)
            def _(t):
                cid = s + t * _NS

                @pl.when(cid < nzch)
                def _():
                    r0 = cid * _SK
                    pltpu.sync_copy(acc.at[pl.ds(r0, _SK)], buf2.at[0])
                    pltpu.sync_copy(buf2.at[0], out_hbm.at[c, pl.ds(r0, _SK)])

            plsc.subcore_barrier()

        def both_phases(key_hbm, v_hbm, e_hbm):
            zero_acc()
            scatter(key_hbm, v_hbm)
            drain(outv)
            zero_acc()
            scatter(key_hbm, e_hbm)
            drain(oute)

        @pl.when(c == 0)
        def _():
            both_phases(row_hbm, zsv, zse)

        @pl.when(c == 1)
        def _():
            both_phases(col_hbm, zrv, zre)

    return sk


def _edge_body(ng, ea_ref, xr_ref, xc_ref, we0, we1, we2, cwe,
               a1, a2, a3, b1, c1, c2, b3, w2m, smat, mask16, p16, g2t, cg,
               w2g, ea2_ref, zsv_ref, zse_ref, zrv_ref, zre_ref,
               sze_ref, sse_ref, accz, accs):
    i = pl.program_id(0)

    @pl.when(i == 0)
    def _():
        accz[...] = jnp.zeros_like(accz)
        accs[...] = jnp.zeros_like(accs)

    ea = ea_ref[...]
    xr = xr_ref[...]
    xc = xc_ref[...]
    ea2 = _dot(ea, we0[...]) + _dot(xr, we1[...]) + _dot(xc, we2[...]) + cwe[...]
    ea2_ref[...] = ea2

    ea2a3 = _dot(ea2, a3[...]) + b1[...]
    ea2c2 = _dot(ea2, c2[...]) + b3[...]

    def side(q, kv, v_ref, e_ref):
        h = _leaky(_dot(q, a1[...]) + _dot(kv, a2[...]) + ea2a3)
        e = jnp.exp(_dot(h, w2m[...])) * mask16[...]
        v = _dot(kv, c1[...]) + ea2c2
        v_ref[...] = _dot(e, smat[...]) * v
        e_ref[...] = _dot(e, p16[...])

    side(xr, xc, zsv_ref, zse_ref)
    side(xc, xr, zrv_ref, zre_ref)

    hg = _leaky(_dot(ea2, g2t[...]) + cg[...])
    eg = jnp.exp(_dot(hg, w2g[...])) * mask16[...]
    accz[...] += _dot_t(eg, ea2)
    accs[...] += jnp.sum(eg, axis=0, keepdims=True)

    @pl.when(i == ng - 1)
    def _():
        sze_ref[...] = accz[...]
        sse_ref[...] = accs[...]


def _edge_call(E, D, EB):
    ng = E // EB
    blk = lambda r, c: pl.BlockSpec((r, c), lambda i: (i, 0))
    full = lambda r, c: pl.BlockSpec((r, c), lambda i: (0, 0))
    return pl.pallas_call(
        functools.partial(_edge_body, ng),
        grid=(ng,),
        in_specs=[blk(EB, D), blk(EB, D), blk(EB, D),
                  full(D, D), full(D, D), full(D, D), full(1, D),
                  full(D, D), full(D, D), full(D, D), full(1, D),
                  full(D, D), full(D, D), full(1, D),
                  full(D, 16), full(16, D), full(1, 16), full(16, D),
                  full(D, D), full(1, D), full(D, 16)],
        out_specs=[blk(EB, D), blk(EB, D), blk(EB, D), blk(EB, D),
                   blk(EB, D), full(16, D), full(1, 16)],
        out_shape=[jax.ShapeDtypeStruct((E, D), F32),
                   jax.ShapeDtypeStruct((E, D), F32),
                   jax.ShapeDtypeStruct((E, D), F32),
                   jax.ShapeDtypeStruct((E, D), F32),
                   jax.ShapeDtypeStruct((E, D), F32),
                   jax.ShapeDtypeStruct((16, D), F32),
                   jax.ShapeDtypeStruct((1, 16), F32)],
        scratch_shapes=[pltpu.VMEM((16, D), F32), pltpu.VMEM((1, 16), F32)],
        compiler_params=pltpu.CompilerParams(
            dimension_semantics=("arbitrary",)),
    )


def _node_body(ng, x_ref, sv_ref, se_ref, rv_ref, re_ref,
               wn1, wn2, wn3, cwn, gn2, cgn, w2n, smat, smat128, mask16,
               sze_ref, sse_ref, u_ref, wg1, wg2, wg3, bg,
               x2_ref, u2_ref, accz, accs):
    i = pl.program_id(0)

    @pl.when(i == 0)
    def _():
        accz[...] = jnp.zeros_like(accz)
        accs[...] = jnp.zeros_like(accs)

    sm = smat[...]
    sm128 = smat128[...]
    sent = sv_ref[...] / (_dot(se_ref[...], sm128) + 1e-16)
    recv = rv_ref[...] / (_dot(re_ref[...], sm128) + 1e-16)
    x2 = (_dot(x_ref[...], wn1[...]) + _dot(sent, wn2[...])
          + _dot(recv, wn3[...]) + cwn[...])
    x2_ref[...] = x2

    hn = _leaky(_dot(x2, gn2[...]) + cgn[...])
    en = jnp.exp(_dot(hn, w2n[...])) * mask16[...]
    accz[...] += _dot_t(en, x2)
    accs[...] += jnp.sum(en, axis=0, keepdims=True)

    @pl.when(i == ng - 1)
    def _():
        naz = jnp.sum(accz[...] * sm, axis=0, keepdims=True)
        nad = _dot(accs[...], sm) + 1e-16
        na = naz / nad
        eaz = jnp.sum(sze_ref[...] * sm, axis=0, keepdims=True)
        ead = _dot(sse_ref[...], sm) + 1e-16
        eav = eaz / ead
        u2_ref[...] = (_dot(u_ref[...], wg1[...]) + _dot(na, wg2[...])
                       + _dot(eav, wg3[...]) + bg[...])


def _node_call(N, D, NB):
    ng = N // NB
    blk = lambda r, c: pl.BlockSpec((r, c), lambda i: (i, 0))
    full = lambda r, c: pl.BlockSpec((r, c), lambda i: (0, 0))
    return pl.pallas_call(
        functools.partial(_node_body, ng),
        grid=(ng,),
        in_specs=[blk(NB, D), blk(NB, D), blk(NB, D), blk(NB, D), blk(NB, D),
                  full(D, D), full(D, D), full(D, D), full(1, D),
                  full(D, D), full(1, D), full(D, 16), full(16, D),
                  full(D, D), full(1, 16), full(16, D), full(1, 16),
                  full(1, D), full(D, D), full(D, D), full(D, D),
                  full(1, D)],
        out_specs=[blk(NB, D), full(1, D)],
        out_shape=[jax.ShapeDtypeStruct((N, D), F32),
                   jax.ShapeDtypeStruct((1, D), F32)],
        scratch_shapes=[pltpu.VMEM((16, D), F32), pltpu.VMEM((1, 16), F32)],
        compiler_params=pltpu.CompilerParams(
            dimension_semantics=("arbitrary",)),
    )


def _headmats(D, H, w2):
    HD = D // H
    w2m = jnp.zeros((D, 16), F32)
    for h in range(H):
        w2m = w2m.at[h * HD:(h + 1) * HD, h].set(w2[h])
    return w2m


def kernel(x, edge_index, edge_attr, u, edge_batch, node_batch, num_nodes,
           num_edges, We, be, Wn, bn, Wg, bg, na_W1, na_b1, na_w2, na_W3,
           na_b3, gn_W1, gn_b1, gn_w2, ge_W1, ge_b1, ge_w2):
    N, D = x.shape
    E = edge_attr.shape[0]
    H = na_w2.shape[0]
    HD = D // H

    row = edge_index[0]
    col = edge_index[1]

    sp = np.zeros((16, D), np.float32)
    for h in range(H):
        sp[h, h * HD:(h + 1) * HD] = 1.0
    smat = jnp.asarray(sp)
    smat128 = jnp.concatenate([smat, jnp.zeros((D - 16, D), F32)], axis=0)
    p16 = jnp.concatenate([jnp.eye(16, dtype=F32),
                           jnp.zeros((16, D - 16), F32)], axis=1)
    mask16 = jnp.asarray((np.arange(16) < H).astype(np.float32))[None, :]

    def splitT(W, k):
        return [W[:, i * D:(i + 1) * D].T for i in range(k)]

    we = splitT(We, 4)
    cwe = (u @ we[3] + be).reshape(1, D)
    a1, a2, a3 = splitT(na_W1, 3)
    c1, c2 = splitT(na_W3, 2)
    w2m = _headmats(D, H, na_w2)
    g1, g2 = splitT(ge_W1, 2)
    cg = (u @ g1 + ge_b1).reshape(1, D)
    w2g = _headmats(D, H, ge_w2)

    xr, xc = _gather_call(N, E, D)(x, row, col)

    EB = 640
    ea2, zsv, zse, zrv, zre, sze, sse = _edge_call(E, D, EB)(
        edge_attr, xr, xc, we[0], we[1], we[2], cwe,
        a1, a2, a3, na_b1.reshape(1, D), c1, c2, na_b3.reshape(1, D),
        w2m, smat, mask16, p16, g2, cg, w2g)

    zv0 = jnp.zeros((_SK, D), F32)
    accv, acce = _scatter_call(N, E, D)(zsv, zse, zrv, zre, row, col, zv0)

    wn1, wn2, wn3, wn4 = splitT(Wn, 4)
    cwn = (u @ wn4 + bn).reshape(1, D)
    gn1, gn2 = splitT(gn_W1, 2)
    cgn = (u @ gn1 + gn_b1).reshape(1, D)
    w2n = _headmats(D, H, gn_w2)
    wg1, wg2, wg3 = splitT(Wg, 3)

    NB = 1000
    x2, u2 = _node_call(N, D, NB)(
        x, accv[0], acce[0], accv[1], acce[1],
        wn1, wn2, wn3, cwn, gn2, cgn, w2n, smat, smat128, mask16,
        sze, sse, u.reshape(1, D), wg1, wg2, wg3, bg.reshape(1, D))

    return (x2, ea2, u2)
